# baseline (speedup 1.0000x reference)
import functools
import math

import jax
import jax.numpy as jnp
from jax.experimental import pallas as pl
from jax.experimental.pallas import tpu as pltpu


def _fused_kernel(x_hbm, wq_hbm, wk_hbm, wv_hbm, o_ref,
                  xv_ref, wq_ref, wk_ref, wv_ref, sems,
                  *, B, T, N, H, d_head, scale):
    D = x_hbm.shape[3]
    R = B * T * N
    S = B * N
    Dh = D // 2

    copies = []
    idx = 0
    for src, dst in ((wq_hbm, wq_ref), (wk_hbm, wk_ref), (wv_hbm, wv_ref)):
        for h in range(2):
            sl = pl.ds(h * Dh, Dh)
            cp = pltpu.make_async_copy(src.at[sl], dst.at[sl], sems.at[idx])
            cp.start()
            copies.append(cp)
            idx += 1
    cx = pltpu.make_async_copy(x_hbm, xv_ref, sems.at[idx])
    cx.start()

    cx.wait()
    x = xv_ref[...]
    m = jnp.max(x, axis=1, keepdims=True)
    e = jnp.exp(x - m)
    ta = jnp.sum(x * e, axis=1) / jnp.sum(e, axis=1)
    tb = ta.reshape(S, D).astype(jnp.bfloat16)
    xb = x.reshape(R, D).astype(jnp.bfloat16)

    def proj(rows, w_ref, w_copies):
        w_copies[0].wait()
        w0 = w_ref[pl.ds(0, Dh), :].astype(jnp.bfloat16)
        acc = jnp.dot(rows[:, :Dh], w0,
                      preferred_element_type=jnp.float32)
        w_copies[1].wait()
        w1 = w_ref[pl.ds(Dh, Dh), :].astype(jnp.bfloat16)
        return acc + jnp.dot(rows[:, Dh:], w1,
                             preferred_element_type=jnp.float32)

    q = (proj(xb, wq_ref, copies[0:2]) * scale).astype(jnp.bfloat16)
    k = proj(tb, wk_ref, copies[2:4])
    v = proj(tb, wv_ref, copies[4:6])

    C = B * H * N
    rh = (jax.lax.broadcasted_iota(jnp.int32, (C, D), 0) % (H * N)) // N
    lh = jax.lax.broadcasted_iota(jnp.int32, (C, D), 1) // d_head
    hmask = rh == lh
    kb = jnp.broadcast_to(k.reshape(B, 1, N, D), (B, H, N, D)).reshape(C, D)
    vb = jnp.broadcast_to(v.reshape(B, 1, N, D), (B, H, N, D)).reshape(C, D)
    zero = jnp.zeros((), jnp.bfloat16)
    kbig = jnp.where(hmask, kb.astype(jnp.bfloat16), zero)
    vbig = jnp.where(hmask, vb.astype(jnp.bfloat16), zero)

    s = jax.lax.dot_general(q, kbig, (((1,), (1,)), ((), ())),
                            preferred_element_type=jnp.float32)
    rb = jax.lax.broadcasted_iota(jnp.int32, (R, C), 0) // (T * N)
    cb = jax.lax.broadcasted_iota(jnp.int32, (R, C), 1) // (H * N)
    p = jnp.where(rb == cb, jnp.maximum(s, 0.0), 0.0).astype(jnp.bfloat16)

    o = jnp.dot(p, vbig, preferred_element_type=jnp.float32)
    o = o.reshape(B, T, N, D) + v.reshape(B, 1, N, D)
    o_ref[...] = o.astype(o_ref.dtype)


def kernel(x, boxes_in_flat, wq, wk, wv):
    del boxes_in_flat
    B, T, N, D = x.shape
    H = 8
    d_head = D // H
    scale = 1.0 / math.sqrt(d_head)

    kern = functools.partial(
        _fused_kernel, B=B, T=T, N=N, H=H, d_head=d_head, scale=scale)
    return pl.pallas_call(
        kern,
        out_shape=jax.ShapeDtypeStruct((B, T, N, D), x.dtype),
        in_specs=[
            pl.BlockSpec(memory_space=pl.ANY),
            pl.BlockSpec(memory_space=pl.ANY),
            pl.BlockSpec(memory_space=pl.ANY),
            pl.BlockSpec(memory_space=pl.ANY),
        ],
        scratch_shapes=[
            pltpu.VMEM((B, T, N, D), jnp.float32),
            pltpu.VMEM((D, D), jnp.float32),
            pltpu.VMEM((D, D), jnp.float32),
            pltpu.VMEM((D, D), jnp.float32),
            pltpu.SemaphoreType.DMA((7,)),
        ],
    )(x, wq, wk, wv)


# speedup vs baseline: 1.3648x; 1.3648x over previous
import functools
import math

import jax
import jax.numpy as jnp
from jax.experimental import pallas as pl
from jax.experimental.pallas import tpu as pltpu


def _fused_kernel(x_hbm, wq_hbm, wk_hbm, wv_hbm, o_ref,
                  xv_ref, wq_ref, wk_ref, wv_ref, sems,
                  *, B, T, N, H, d_head, scale):
    D = x_hbm.shape[3]
    R = B * T * N
    S = B * N
    Dh = D // 2

    cx = pltpu.make_async_copy(x_hbm, xv_ref, sems.at[6])
    cx.start()
    copies = []
    idx = 0
    for src, dst in ((wq_hbm, wq_ref), (wk_hbm, wk_ref), (wv_hbm, wv_ref)):
        for h in range(2):
            sl = pl.ds(h * Dh, Dh)
            cp = pltpu.make_async_copy(src.at[sl], dst.at[sl], sems.at[idx])
            cp.start()
            copies.append(cp)
            idx += 1

    cx.wait()
    x = xv_ref[...]
    m = jnp.max(x, axis=1, keepdims=True)
    e = jnp.exp(x - m)
    ta = jnp.sum(x * e, axis=1) / jnp.sum(e, axis=1)
    tb = ta.reshape(S, D).astype(jnp.bfloat16)
    xb = x.reshape(R, D).astype(jnp.bfloat16)

    def proj(rows, w_ref, w_copies):
        w_copies[0].wait()
        w0 = w_ref[pl.ds(0, Dh), :].astype(jnp.bfloat16)
        acc = jnp.dot(rows[:, :Dh], w0,
                      preferred_element_type=jnp.float32)
        w_copies[1].wait()
        w1 = w_ref[pl.ds(Dh, Dh), :].astype(jnp.bfloat16)
        return acc + jnp.dot(rows[:, Dh:], w1,
                             preferred_element_type=jnp.float32)

    q = (proj(xb, wq_ref, copies[0:2]) * scale).astype(jnp.bfloat16)
    k = proj(tb, wk_ref, copies[2:4])
    v = proj(tb, wv_ref, copies[4:6])

    C = B * H * N
    rh = (jax.lax.broadcasted_iota(jnp.int32, (C, D), 0) % (H * N)) // N
    lh = jax.lax.broadcasted_iota(jnp.int32, (C, D), 1) // d_head
    hmask = rh == lh
    kb = jnp.broadcast_to(k.reshape(B, 1, N, D), (B, H, N, D)).reshape(C, D)
    vb = jnp.broadcast_to(v.reshape(B, 1, N, D), (B, H, N, D)).reshape(C, D)
    zero = jnp.zeros((), jnp.bfloat16)
    kbig = jnp.where(hmask, kb.astype(jnp.bfloat16), zero)
    vbig = jnp.where(hmask, vb.astype(jnp.bfloat16), zero)

    s = jax.lax.dot_general(q, kbig, (((1,), (1,)), ((), ())),
                            preferred_element_type=jnp.float32)
    rb = jax.lax.broadcasted_iota(jnp.int32, (R, C), 0) // (T * N)
    cb = jax.lax.broadcasted_iota(jnp.int32, (R, C), 1) // (H * N)
    p = jnp.where(rb == cb, jnp.maximum(s, 0.0), 0.0).astype(jnp.bfloat16)

    o = jnp.dot(p, vbig, preferred_element_type=jnp.float32)
    o = o.reshape(B, T, N, D) + v.reshape(B, 1, N, D)
    o_ref[...] = o.astype(o_ref.dtype)


def kernel(x, boxes_in_flat, wq, wk, wv):
    del boxes_in_flat
    B, T, N, D = x.shape
    H = 8
    d_head = D // H
    scale = 1.0 / math.sqrt(d_head)

    kern = functools.partial(
        _fused_kernel, B=B, T=T, N=N, H=H, d_head=d_head, scale=scale)
    return pl.pallas_call(
        kern,
        out_shape=jax.ShapeDtypeStruct((B, T, N, D), x.dtype),
        in_specs=[
            pl.BlockSpec(memory_space=pl.ANY),
            pl.BlockSpec(memory_space=pl.ANY),
            pl.BlockSpec(memory_space=pl.ANY),
            pl.BlockSpec(memory_space=pl.ANY),
        ],
        scratch_shapes=[
            pltpu.VMEM((B, T, N, D), jnp.float32),
            pltpu.VMEM((D, D), jnp.float32),
            pltpu.VMEM((D, D), jnp.float32),
            pltpu.VMEM((D, D), jnp.float32),
            pltpu.SemaphoreType.DMA((7,)),
        ],
    )(x, wq, wk, wv)


# speedup vs baseline: 1.3768x; 1.0088x over previous
import functools
import math

import jax
import jax.numpy as jnp
from jax.experimental import pallas as pl
from jax.experimental.pallas import tpu as pltpu


def _fused_kernel(x_hbm, wq_hbm, wk_hbm, wv_hbm, o_ref,
                  xv_ref, wq_ref, wk_ref, wv_ref, sems,
                  *, B, T, N, H, d_head, scale):
    D = x_hbm.shape[3]
    R = B * T * N
    S = B * N

    NC = max(1, D // 256)
    KC = D // NC

    cx = pltpu.make_async_copy(x_hbm, xv_ref, sems.at[3 * NC])
    cx.start()
    copies = []
    idx = 0
    for src, dst in ((wq_hbm, wq_ref), (wk_hbm, wk_ref), (wv_hbm, wv_ref)):
        for c in range(NC):
            sl = pl.ds(c * KC, KC)
            cp = pltpu.make_async_copy(src.at[sl], dst.at[sl], sems.at[idx])
            cp.start()
            copies.append(cp)
            idx += 1

    cx.wait()
    x = xv_ref[...]
    m = jnp.max(x, axis=1, keepdims=True)
    e = jnp.exp(x - m)
    ta = jnp.sum(x * e, axis=1) / jnp.sum(e, axis=1)
    tb = ta.reshape(S, D).astype(jnp.bfloat16)
    xb = x.reshape(R, D).astype(jnp.bfloat16)

    def proj(rows, w_ref, w_copies):
        acc = None
        for c in range(NC):
            w_copies[c].wait()
            wc = w_ref[pl.ds(c * KC, KC), :].astype(jnp.bfloat16)
            part = jnp.dot(rows[:, c * KC:(c + 1) * KC], wc,
                           preferred_element_type=jnp.float32)
            acc = part if acc is None else acc + part
        return acc

    q = (proj(xb, wq_ref, copies[0:NC]) * scale).astype(jnp.bfloat16)
    k = proj(tb, wk_ref, copies[NC:2 * NC])
    v = proj(tb, wv_ref, copies[2 * NC:3 * NC])

    C = B * H * N
    rh = (jax.lax.broadcasted_iota(jnp.int32, (C, D), 0) % (H * N)) // N
    lh = jax.lax.broadcasted_iota(jnp.int32, (C, D), 1) // d_head
    hmask = rh == lh
    kb = jnp.broadcast_to(k.reshape(B, 1, N, D), (B, H, N, D)).reshape(C, D)
    vb = jnp.broadcast_to(v.reshape(B, 1, N, D), (B, H, N, D)).reshape(C, D)
    zero = jnp.zeros((), jnp.bfloat16)
    kbig = jnp.where(hmask, kb.astype(jnp.bfloat16), zero)
    vbig = jnp.where(hmask, vb.astype(jnp.bfloat16), zero)

    s = jax.lax.dot_general(q, kbig, (((1,), (1,)), ((), ())),
                            preferred_element_type=jnp.float32)
    rb = jax.lax.broadcasted_iota(jnp.int32, (R, C), 0) // (T * N)
    cb = jax.lax.broadcasted_iota(jnp.int32, (R, C), 1) // (H * N)
    p = jnp.where(rb == cb, jnp.maximum(s, 0.0), 0.0).astype(jnp.bfloat16)

    o = jnp.dot(p, vbig, preferred_element_type=jnp.float32)
    o = o.reshape(B, T, N, D) + v.reshape(B, 1, N, D)
    o_ref[...] = o.astype(o_ref.dtype)


def kernel(x, boxes_in_flat, wq, wk, wv):
    del boxes_in_flat
    B, T, N, D = x.shape
    H = 8
    d_head = D // H
    scale = 1.0 / math.sqrt(d_head)

    kern = functools.partial(
        _fused_kernel, B=B, T=T, N=N, H=H, d_head=d_head, scale=scale)
    return pl.pallas_call(
        kern,
        out_shape=jax.ShapeDtypeStruct((B, T, N, D), x.dtype),
        in_specs=[
            pl.BlockSpec(memory_space=pl.ANY),
            pl.BlockSpec(memory_space=pl.ANY),
            pl.BlockSpec(memory_space=pl.ANY),
            pl.BlockSpec(memory_space=pl.ANY),
        ],
        scratch_shapes=[
            pltpu.VMEM((B, T, N, D), jnp.float32),
            pltpu.VMEM((D, D), jnp.float32),
            pltpu.VMEM((D, D), jnp.float32),
            pltpu.VMEM((D, D), jnp.float32),
            pltpu.SemaphoreType.DMA((3 * max(1, D // 256) + 1,)),
        ],
    )(x, wq, wk, wv)
